# baseline (speedup 1.0000x reference)
import functools

import jax
import jax.numpy as jnp
from jax import lax
from jax.experimental import pallas as pl
from jax.experimental.pallas import tpu as pltpu
from jax.experimental.pallas import tpu_sc as plsc


def _sc_gather(table, input_id):
    B = input_id.shape[0]
    D = table.shape[1]
    info = plsc.get_sparse_core_info()
    NW = info.num_cores * info.num_subcores
    b_per_w = B // NW
    mesh = plsc.VectorSubcoreMesh(core_axis_name="c", subcore_axis_name="s")

    @functools.partial(
        pl.kernel,
        mesh=mesh,
        out_type=jax.ShapeDtypeStruct((B, D), jnp.float32),
        scratch_types=[
            pltpu.VMEM((b_per_w,), jnp.int32),
            pltpu.VMEM((b_per_w, D), jnp.float32),
            pltpu.SemaphoreType.DMA,
        ],
        compiler_params=pltpu.CompilerParams(use_tc_tiling_on_sc=False),
    )
    def gather_kernel(table_hbm, idx_hbm, out_hbm, idx_v, rows_v, sem):
        wid = lax.axis_index("s") * info.num_cores + lax.axis_index("c")
        base = wid * b_per_w
        pltpu.sync_copy(idx_hbm.at[pl.ds(base, b_per_w)], idx_v)
        pltpu.async_copy(table_hbm.at[idx_v], rows_v, sem).wait()
        pltpu.sync_copy(rows_v, out_hbm.at[pl.ds(base, b_per_w)])

    return gather_kernel(table, input_id.astype(jnp.int32))


_VB = 2048


def _classifier_body(vec_ref, wc_ref, bc_ref, wi_ref, bi_ref, cat_ref, id_ref):
    @pl.when(pl.program_id(0) == 0)
    def _():
        cat_ref[...] = (
            lax.dot_general(
                vec_ref[...], wc_ref[...],
                (((1,), (1,)), ((), ())),
                preferred_element_type=jnp.float32,
            )
            + bc_ref[...]
        )

    id_ref[...] = (
        lax.dot_general(
            vec_ref[...], wi_ref[...],
            (((1,), (1,)), ((), ())),
            preferred_element_type=jnp.float32,
        )
        + bi_ref[...]
    )


def _tc_classifiers(vec, Wc, bc, Wi, bi):
    B, D = vec.shape
    C = Wc.shape[0]
    V = Wi.shape[0]
    nblk = pl.cdiv(V, _VB)
    return pl.pallas_call(
        _classifier_body,
        grid=(nblk,),
        in_specs=[
            pl.BlockSpec((B, D), lambda j: (0, 0)),
            pl.BlockSpec((C, D), lambda j: (0, 0)),
            pl.BlockSpec((1, C), lambda j: (0, 0)),
            pl.BlockSpec((_VB, D), lambda j: (j, 0)),
            pl.BlockSpec((1, _VB), lambda j: (0, j)),
        ],
        out_specs=[
            pl.BlockSpec((B, C), lambda j: (0, 0)),
            pl.BlockSpec((B, _VB), lambda j: (0, j)),
        ],
        out_shape=[
            jax.ShapeDtypeStruct((B, C), jnp.float32),
            jax.ShapeDtypeStruct((B, V), jnp.float32),
        ],
    )(vec, Wc, bc.reshape(1, C), Wi, bi.reshape(1, V))


def kernel(table, Wc, bc, Wi, bi, input_id):
    vec = _sc_gather(table, input_id)
    out_category, out_identity = _tc_classifiers(vec, Wc, bc, Wi, bi)
    return (vec, out_category, out_identity)


# speedup vs baseline: 1.0005x; 1.0005x over previous
import functools

import jax
import jax.numpy as jnp
from jax import lax
from jax.experimental import pallas as pl
from jax.experimental.pallas import tpu as pltpu
from jax.experimental.pallas import tpu_sc as plsc


def _sc_gather(table, input_id):
    B = input_id.shape[0]
    D = table.shape[1]
    info = plsc.get_sparse_core_info()
    NW = info.num_cores * info.num_subcores
    b_per_w = B // NW
    mesh = plsc.VectorSubcoreMesh(core_axis_name="c", subcore_axis_name="s")

    @functools.partial(
        pl.kernel,
        mesh=mesh,
        out_type=jax.ShapeDtypeStruct((B, D), jnp.float32),
        scratch_types=[
            pltpu.VMEM((b_per_w,), jnp.int32),
            pltpu.VMEM((b_per_w, D), jnp.float32),
            pltpu.SemaphoreType.DMA,
        ],
        compiler_params=pltpu.CompilerParams(use_tc_tiling_on_sc=False),
    )
    def gather_kernel(table_hbm, idx_hbm, out_hbm, idx_v, rows_v, sem):
        wid = lax.axis_index("s") * info.num_cores + lax.axis_index("c")
        base = wid * b_per_w
        pltpu.sync_copy(idx_hbm.at[pl.ds(base, b_per_w)], idx_v)
        pltpu.async_copy(table_hbm.at[idx_v], rows_v, sem).wait()
        pltpu.sync_copy(rows_v, out_hbm.at[pl.ds(base, b_per_w)])

    return gather_kernel(table, input_id.astype(jnp.int32))


_VB = 2048
_NBUF = 4


def _classifier_body(vec_ref, wc_ref, bc_ref, wi_ref, bi_ref, cat_ref, id_hbm,
                     id_buf, id_tail, sems, tail_sem):
    j = pl.program_id(0)
    nblk = pl.num_programs(0)
    V = id_hbm.shape[1]
    tail = V - (nblk - 1) * _VB
    slot = lax.rem(j, _NBUF)

    @pl.when(j == 0)
    def _():
        cat_ref[...] = (
            lax.dot_general(
                vec_ref[...], wc_ref[...],
                (((1,), (1,)), ((), ())),
                preferred_element_type=jnp.float32,
            )
            + bc_ref[...]
        )

    logits = (
        lax.dot_general(
            vec_ref[...], wi_ref[...],
            (((1,), (1,)), ((), ())),
            preferred_element_type=jnp.float32,
        )
        + bi_ref[...]
    )

    @pl.when(j < nblk - 1)
    def _():
        for k in range(_NBUF):
            @pl.when(slot == k)
            def _():
                @pl.when(j >= _NBUF)
                def _():
                    pltpu.make_async_copy(
                        id_buf.at[k], id_hbm.at[:, pl.ds(0, _VB)], sems.at[k]
                    ).wait()

                id_buf[k] = logits
                pltpu.make_async_copy(
                    id_buf.at[k], id_hbm.at[:, pl.ds(j * _VB, _VB)], sems.at[k]
                ).start()

    @pl.when(j == nblk - 1)
    def _():
        id_tail[...] = logits[:, :tail]
        pltpu.make_async_copy(
            id_tail, id_hbm.at[:, pl.ds((nblk - 1) * _VB, tail)], tail_sem
        ).start()
        pltpu.make_async_copy(
            id_tail, id_hbm.at[:, pl.ds((nblk - 1) * _VB, tail)], tail_sem
        ).wait()
        for k in range(_NBUF):
            pltpu.make_async_copy(
                id_buf.at[k], id_hbm.at[:, pl.ds(0, _VB)], sems.at[k]
            ).wait()


def _tc_classifiers(vec, Wc, bc, Wi, bi):
    B, D = vec.shape
    C = Wc.shape[0]
    V = Wi.shape[0]
    nblk = pl.cdiv(V, _VB)
    return pl.pallas_call(
        _classifier_body,
        grid=(nblk,),
        in_specs=[
            pl.BlockSpec((B, D), lambda j: (0, 0)),
            pl.BlockSpec((C, D), lambda j: (0, 0)),
            pl.BlockSpec((1, C), lambda j: (0, 0)),
            pl.BlockSpec((_VB, D), lambda j: (j, 0)),
            pl.BlockSpec((1, _VB), lambda j: (0, j)),
        ],
        out_specs=[
            pl.BlockSpec((B, C), lambda j: (0, 0)),
            pl.BlockSpec(memory_space=pl.ANY),
        ],
        out_shape=[
            jax.ShapeDtypeStruct((B, C), jnp.float32),
            jax.ShapeDtypeStruct((B, V), jnp.float32),
        ],
        scratch_shapes=[
            pltpu.VMEM((_NBUF, B, _VB), jnp.float32),
            pltpu.VMEM((B, V - (nblk - 1) * _VB), jnp.float32),
            pltpu.SemaphoreType.DMA((_NBUF,)),
            pltpu.SemaphoreType.DMA,
        ],
    )(vec, Wc, bc.reshape(1, C), Wi, bi.reshape(1, V))


def kernel(table, Wc, bc, Wi, bi, input_id):
    vec = _sc_gather(table, input_id)
    out_category, out_identity = _tc_classifiers(vec, Wc, bc, Wi, bi)
    return (vec, out_category, out_identity)


# speedup vs baseline: 2.1364x; 2.1353x over previous
import functools

import jax
import jax.numpy as jnp
from jax import lax
from jax.experimental import pallas as pl
from jax.experimental.pallas import tpu as pltpu
from jax.experimental.pallas import tpu_sc as plsc


def _sc_gather(table, input_id):
    B = input_id.shape[0]
    D = table.shape[1]
    info = plsc.get_sparse_core_info()
    NW = info.num_cores * info.num_subcores
    b_per_w = B // NW
    mesh = plsc.VectorSubcoreMesh(core_axis_name="c", subcore_axis_name="s")

    @functools.partial(
        pl.kernel,
        mesh=mesh,
        out_type=jax.ShapeDtypeStruct((B, D), jnp.float32),
        scratch_types=[
            pltpu.VMEM((b_per_w,), jnp.int32),
            pltpu.VMEM((b_per_w, D), jnp.float32),
            pltpu.SemaphoreType.DMA,
        ],
        compiler_params=pltpu.CompilerParams(use_tc_tiling_on_sc=False),
    )
    def gather_kernel(table_hbm, idx_hbm, out_hbm, idx_v, rows_v, sem):
        wid = lax.axis_index("s") * info.num_cores + lax.axis_index("c")
        base = wid * b_per_w
        pltpu.sync_copy(idx_hbm.at[pl.ds(base, b_per_w)], idx_v)
        pltpu.async_copy(table_hbm.at[idx_v], rows_v, sem).wait()
        pltpu.sync_copy(rows_v, out_hbm.at[pl.ds(base, b_per_w)])

    return gather_kernel(table, input_id.astype(jnp.int32))


_VB = 2048


def _contract(lhs, rhs):
    return lax.dot_general(
        lhs, rhs, (((0,), (1,)), ((), ())), preferred_element_type=jnp.float32
    )


def _classifier_body(vec_ref, wct_ref, bc_ref, wit_ref, bi_ref,
                     vect_ref, catt_ref, idt_ref):
    @pl.when(pl.program_id(0) == 0)
    def _():
        D = vec_ref.shape[1]
        rows = lax.broadcasted_iota(jnp.int32, (D, D), 0)
        cols = lax.broadcasted_iota(jnp.int32, (D, D), 1)
        eye = (rows == cols).astype(jnp.float32)
        vect_ref[...] = _contract(eye, vec_ref[...])
        catt_ref[...] = _contract(wct_ref[...], vec_ref[...]) + bc_ref[...]

    idt_ref[...] = _contract(wit_ref[...], vec_ref[...]) + bi_ref[0]


def _tc_classifiers(vec, Wc, bc, Wi, bi):
    B, D = vec.shape
    C = Wc.shape[0]
    V = Wi.shape[0]
    nblk = pl.cdiv(V, _VB)
    bi_cols = jnp.pad(bi, (0, nblk * _VB - V)).reshape(nblk, _VB, 1)
    vect, catt, idt = pl.pallas_call(
        _classifier_body,
        grid=(nblk,),
        in_specs=[
            pl.BlockSpec((B, D), lambda j: (0, 0)),
            pl.BlockSpec((D, C), lambda j: (0, 0)),
            pl.BlockSpec((C, 1), lambda j: (0, 0)),
            pl.BlockSpec((D, _VB), lambda j: (0, j)),
            pl.BlockSpec((1, _VB, 1), lambda j: (j, 0, 0)),
        ],
        out_specs=[
            pl.BlockSpec((D, B), lambda j: (0, 0)),
            pl.BlockSpec((C, B), lambda j: (0, 0)),
            pl.BlockSpec((_VB, B), lambda j: (j, 0)),
        ],
        out_shape=[
            jax.ShapeDtypeStruct((D, B), jnp.float32),
            jax.ShapeDtypeStruct((C, B), jnp.float32),
            jax.ShapeDtypeStruct((V, B), jnp.float32),
        ],
    )(vec, Wc.T, bc.reshape(C, 1), Wi.T, bi_cols)
    return vect.T, catt.T, idt.T


def kernel(table, Wc, bc, Wi, bi, input_id):
    vec = _sc_gather(table, input_id)
    vec_out, out_category, out_identity = _tc_classifiers(vec, Wc, bc, Wi, bi)
    return (vec_out, out_category, out_identity)


# speedup vs baseline: 2.8161x; 1.3181x over previous
import functools

import jax
import jax.numpy as jnp
from jax import lax
from jax.experimental import pallas as pl
from jax.experimental.pallas import tpu as pltpu
from jax.experimental.pallas import tpu_sc as plsc


def _sc_gather(table, input_id):
    B = input_id.shape[0]
    D = table.shape[1]
    info = plsc.get_sparse_core_info()
    NW = info.num_cores * info.num_subcores
    b_per_w = B // NW
    mesh = plsc.VectorSubcoreMesh(core_axis_name="c", subcore_axis_name="s")

    @functools.partial(
        pl.kernel,
        mesh=mesh,
        out_type=jax.ShapeDtypeStruct((B, D), jnp.float32),
        scratch_types=[
            pltpu.VMEM((b_per_w,), jnp.int32),
            pltpu.VMEM((b_per_w, D), jnp.float32),
            pltpu.SemaphoreType.DMA,
        ],
        compiler_params=pltpu.CompilerParams(use_tc_tiling_on_sc=False),
    )
    def gather_kernel(table_hbm, idx_hbm, out_hbm, idx_v, rows_v, sem):
        wid = lax.axis_index("s") * info.num_cores + lax.axis_index("c")
        base = wid * b_per_w
        pltpu.sync_copy(idx_hbm.at[pl.ds(base, b_per_w)], idx_v)
        pltpu.async_copy(table_hbm.at[idx_v], rows_v, sem).wait()
        pltpu.sync_copy(rows_v, out_hbm.at[pl.ds(base, b_per_w)])

    return gather_kernel(table, input_id.astype(jnp.int32))


_VB = 2048


def _contract(lhs, rhs):
    return lax.dot_general(
        lhs, rhs, (((0,), (1,)), ((), ())), preferred_element_type=jnp.float32
    )


def _classifier_body(vec_ref, wct_ref, bc_ref, wit_ref, bi_ref,
                     vect_ref, catt_ref, idt_ref):
    @pl.when(pl.program_id(0) == 0)
    def _():
        D = vec_ref.shape[1]
        rows = lax.broadcasted_iota(jnp.int32, (D, D), 0)
        cols = lax.broadcasted_iota(jnp.int32, (D, D), 1)
        eye = (rows == cols).astype(jnp.float32)
        vect_ref[...] = _contract(eye, vec_ref[...])
        catt_ref[...] = _contract(wct_ref[...], vec_ref[...]) + bc_ref[...]

    bi_col = jnp.swapaxes(bi_ref[...].reshape(1, -1), 0, 1)
    idt_ref[...] = _contract(wit_ref[...], vec_ref[...]) + bi_col


def _tc_classifiers(vec, Wc, bc, Wi, bi):
    B, D = vec.shape
    C = Wc.shape[0]
    V = Wi.shape[0]
    nblk = pl.cdiv(V, _VB)
    vect, catt, idt = pl.pallas_call(
        _classifier_body,
        grid=(nblk,),
        in_specs=[
            pl.BlockSpec((B, D), lambda j: (0, 0)),
            pl.BlockSpec((D, C), lambda j: (0, 0)),
            pl.BlockSpec((C, 1), lambda j: (0, 0)),
            pl.BlockSpec((D, _VB), lambda j: (0, j)),
            pl.BlockSpec((_VB,), lambda j: (j,)),
        ],
        out_specs=[
            pl.BlockSpec((D, B), lambda j: (0, 0)),
            pl.BlockSpec((C, B), lambda j: (0, 0)),
            pl.BlockSpec((_VB, B), lambda j: (j, 0)),
        ],
        out_shape=[
            jax.ShapeDtypeStruct((D, B), jnp.float32),
            jax.ShapeDtypeStruct((C, B), jnp.float32),
            jax.ShapeDtypeStruct((V, B), jnp.float32),
        ],
    )(vec, Wc.T, bc.reshape(C, 1), Wi.T, bi)
    return vect.T, catt.T, idt.T


def kernel(table, Wc, bc, Wi, bi, input_id):
    vec = _sc_gather(table, input_id)
    vec_out, out_category, out_identity = _tc_classifiers(vec, Wc, bc, Wi, bi)
    return (vec_out, out_category, out_identity)


# speedup vs baseline: 2.8648x; 1.0173x over previous
import functools

import jax
import jax.numpy as jnp
from jax import lax
from jax.experimental import pallas as pl
from jax.experimental.pallas import tpu as pltpu
from jax.experimental.pallas import tpu_sc as plsc


def _sc_gather(table, input_id):
    B = input_id.shape[0]
    D = table.shape[1]
    info = plsc.get_sparse_core_info()
    NW = info.num_cores * info.num_subcores
    b_per_w = B // NW
    mesh = plsc.VectorSubcoreMesh(core_axis_name="c", subcore_axis_name="s")

    @functools.partial(
        pl.kernel,
        mesh=mesh,
        out_type=jax.ShapeDtypeStruct((B, D), jnp.float32),
        scratch_types=[
            pltpu.VMEM((b_per_w,), jnp.int32),
            pltpu.VMEM((b_per_w, D), jnp.float32),
            pltpu.SemaphoreType.DMA,
        ],
        compiler_params=pltpu.CompilerParams(use_tc_tiling_on_sc=False),
    )
    def gather_kernel(table_hbm, idx_hbm, out_hbm, idx_v, rows_v, sem):
        wid = lax.axis_index("s") * info.num_cores + lax.axis_index("c")
        base = wid * b_per_w
        pltpu.sync_copy(idx_hbm.at[pl.ds(base, b_per_w)], idx_v)
        pltpu.async_copy(table_hbm.at[idx_v], rows_v, sem).wait()
        pltpu.sync_copy(rows_v, out_hbm.at[pl.ds(base, b_per_w)])

    return gather_kernel(table, input_id.astype(jnp.int32))


_VB = 4096


def _contract(lhs, rhs):
    return lax.dot_general(
        lhs, rhs, (((0,), (1,)), ((), ())), preferred_element_type=jnp.float32
    )


def _classifier_body(vec_ref, wct_ref, bc_ref, wit_ref, bi_ref,
                     vect_ref, catt_ref, idt_ref):
    @pl.when(pl.program_id(0) == 0)
    def _():
        D = vec_ref.shape[1]
        rows = lax.broadcasted_iota(jnp.int32, (D, D), 0)
        cols = lax.broadcasted_iota(jnp.int32, (D, D), 1)
        eye = (rows == cols).astype(jnp.float32)
        vect_ref[...] = _contract(eye, vec_ref[...])
        catt_ref[...] = _contract(wct_ref[...], vec_ref[...]) + bc_ref[...]

    bi_col = jnp.swapaxes(bi_ref[...].reshape(1, -1), 0, 1)
    idt_ref[...] = _contract(wit_ref[...], vec_ref[...]) + bi_col


def _tc_classifiers(vec, Wc, bc, Wi, bi):
    B, D = vec.shape
    C = Wc.shape[0]
    V = Wi.shape[0]
    nblk = pl.cdiv(V, _VB)
    vect, catt, idt = pl.pallas_call(
        _classifier_body,
        grid=(nblk,),
        in_specs=[
            pl.BlockSpec((B, D), lambda j: (0, 0)),
            pl.BlockSpec((D, C), lambda j: (0, 0)),
            pl.BlockSpec((C, 1), lambda j: (0, 0)),
            pl.BlockSpec((D, _VB), lambda j: (0, j)),
            pl.BlockSpec((_VB,), lambda j: (j,)),
        ],
        out_specs=[
            pl.BlockSpec((D, B), lambda j: (0, 0)),
            pl.BlockSpec((C, B), lambda j: (0, 0)),
            pl.BlockSpec((_VB, B), lambda j: (j, 0)),
        ],
        out_shape=[
            jax.ShapeDtypeStruct((D, B), jnp.float32),
            jax.ShapeDtypeStruct((C, B), jnp.float32),
            jax.ShapeDtypeStruct((V, B), jnp.float32),
        ],
    )(vec, Wc.T, bc.reshape(C, 1), Wi.T, bi)
    return vect.T, catt.T, idt.T


def kernel(table, Wc, bc, Wi, bi, input_id):
    vec = _sc_gather(table, input_id)
    vec_out, out_category, out_identity = _tc_classifiers(vec, Wc, bc, Wi, bi)
    return (vec_out, out_category, out_identity)


# speedup vs baseline: 2.9654x; 1.0351x over previous
import functools

import jax
import jax.numpy as jnp
from jax import lax
from jax.experimental import pallas as pl
from jax.experimental.pallas import tpu as pltpu
from jax.experimental.pallas import tpu_sc as plsc


_TB = 2048


def _linearize_body(x_ref, o_ref):
    y = jnp.swapaxes(x_ref[...], 0, 1)
    y3 = y.reshape(_TB // 2, 2, y.shape[1])
    o_ref[...] = jnp.concatenate([y3[:, 0, :], y3[:, 1, :]], axis=1)


def _tc_table_linearize(table):
    V, D = table.shape
    nblk = pl.cdiv(V, _TB)
    out = pl.pallas_call(
        _linearize_body,
        grid=(nblk,),
        in_specs=[pl.BlockSpec((D, _TB), lambda j: (0, j))],
        out_specs=pl.BlockSpec((_TB // 2, 128), lambda j: (j, 0)),
        out_shape=jax.ShapeDtypeStruct((V // 2, 128), jnp.float32),
    )(table.T)
    return out.reshape(V, D)


def _sc_gather(table, input_id):
    B = input_id.shape[0]
    D = table.shape[1]
    info = plsc.get_sparse_core_info()
    NW = info.num_cores * info.num_subcores
    b_per_w = B // NW
    mesh = plsc.VectorSubcoreMesh(core_axis_name="c", subcore_axis_name="s")

    @functools.partial(
        pl.kernel,
        mesh=mesh,
        out_type=jax.ShapeDtypeStruct((B, D), jnp.float32),
        scratch_types=[
            pltpu.VMEM((b_per_w,), jnp.int32),
            pltpu.VMEM((b_per_w, D), jnp.float32),
            pltpu.SemaphoreType.DMA,
        ],
        compiler_params=pltpu.CompilerParams(use_tc_tiling_on_sc=False),
    )
    def gather_kernel(table_hbm, idx_hbm, out_hbm, idx_v, rows_v, sem):
        wid = lax.axis_index("s") * info.num_cores + lax.axis_index("c")
        base = wid * b_per_w
        pltpu.sync_copy(idx_hbm.at[pl.ds(base, b_per_w)], idx_v)
        pltpu.async_copy(table_hbm.at[idx_v], rows_v, sem).wait()
        pltpu.sync_copy(rows_v, out_hbm.at[pl.ds(base, b_per_w)])

    return gather_kernel(table, input_id.astype(jnp.int32))


_VB = 4096


def _contract(lhs, rhs):
    return lax.dot_general(
        lhs, rhs, (((0,), (1,)), ((), ())), preferred_element_type=jnp.float32
    )


def _classifier_body(vec_ref, wct_ref, bc_ref, wit_ref, bi_ref,
                     vect_ref, catt_ref, idt_ref):
    @pl.when(pl.program_id(0) == 0)
    def _():
        D = vec_ref.shape[1]
        rows = lax.broadcasted_iota(jnp.int32, (D, D), 0)
        cols = lax.broadcasted_iota(jnp.int32, (D, D), 1)
        eye = (rows == cols).astype(jnp.float32)
        vect_ref[...] = _contract(eye, vec_ref[...])
        catt_ref[...] = _contract(wct_ref[...], vec_ref[...]) + bc_ref[...]

    bi_col = jnp.swapaxes(bi_ref[...].reshape(1, -1), 0, 1)
    idt_ref[...] = _contract(wit_ref[...], vec_ref[...]) + bi_col


def _tc_classifiers(vec, Wc, bc, Wi, bi):
    B, D = vec.shape
    C = Wc.shape[0]
    V = Wi.shape[0]
    nblk = pl.cdiv(V, _VB)
    vect, catt, idt = pl.pallas_call(
        _classifier_body,
        grid=(nblk,),
        in_specs=[
            pl.BlockSpec((B, D), lambda j: (0, 0)),
            pl.BlockSpec((D, C), lambda j: (0, 0)),
            pl.BlockSpec((C, 1), lambda j: (0, 0)),
            pl.BlockSpec((D, _VB), lambda j: (0, j)),
            pl.BlockSpec((_VB,), lambda j: (j,)),
        ],
        out_specs=[
            pl.BlockSpec((D, B), lambda j: (0, 0)),
            pl.BlockSpec((C, B), lambda j: (0, 0)),
            pl.BlockSpec((_VB, B), lambda j: (j, 0)),
        ],
        out_shape=[
            jax.ShapeDtypeStruct((D, B), jnp.float32),
            jax.ShapeDtypeStruct((C, B), jnp.float32),
            jax.ShapeDtypeStruct((V, B), jnp.float32),
        ],
    )(vec, Wc.T, bc.reshape(C, 1), Wi.T, bi)
    return vect.T, catt.T, idt.T


def kernel(table, Wc, bc, Wi, bi, input_id):
    vec = _sc_gather(_tc_table_linearize(table), input_id)
    vec_out, out_category, out_identity = _tc_classifiers(vec, Wc, bc, Wi, bi)
    return (vec_out, out_category, out_identity)


# speedup vs baseline: 3.1008x; 1.0457x over previous
import functools

import jax
import jax.numpy as jnp
from jax import lax
from jax.experimental import pallas as pl
from jax.experimental.pallas import tpu as pltpu
from jax.experimental.pallas import tpu_sc as plsc

_TB = 2048


def _widen_body(x_ref, o_ref):
    o_ref[:, 0:64] = jnp.swapaxes(x_ref[...], 0, 1)


def _tc_table_widen(table):
    V, D = table.shape
    nblk = pl.cdiv(V, _TB)
    return pl.pallas_call(
        _widen_body,
        grid=(nblk,),
        in_specs=[pl.BlockSpec((D, _TB), lambda j: (0, j))],
        out_specs=pl.BlockSpec((_TB, 128), lambda j: (j, 0)),
        out_shape=jax.ShapeDtypeStruct((V, 128), jnp.float32),
    )(table.T)


def _sc_gather(table128, input_id):
    B = input_id.shape[0]
    W = table128.shape[1]
    info = plsc.get_sparse_core_info()
    NW = info.num_cores * info.num_subcores
    b_per_w = B // NW
    mesh = plsc.VectorSubcoreMesh(core_axis_name="c", subcore_axis_name="s")

    @functools.partial(
        pl.kernel,
        mesh=mesh,
        out_type=jax.ShapeDtypeStruct((B, W), jnp.float32),
        scratch_types=[
            pltpu.VMEM((b_per_w,), jnp.int32),
            pltpu.VMEM((b_per_w, W), jnp.float32),
            pltpu.SemaphoreType.DMA,
        ],
        compiler_params=pltpu.CompilerParams(use_tc_tiling_on_sc=False),
    )
    def gather_kernel(table_hbm, idx_hbm, out_hbm, idx_v, rows_v, sem):
        wid = lax.axis_index("s") * info.num_cores + lax.axis_index("c")
        base = wid * b_per_w
        pltpu.sync_copy(idx_hbm.at[pl.ds(base, b_per_w)], idx_v)
        pltpu.async_copy(table_hbm.at[idx_v], rows_v, sem).wait()
        pltpu.sync_copy(rows_v, out_hbm.at[pl.ds(base, b_per_w)])

    return gather_kernel(table128, input_id.astype(jnp.int32))


_VB = 4096


def _contract(lhs, rhs):
    return lax.dot_general(
        lhs, rhs, (((0,), (1,)), ((), ())), preferred_element_type=jnp.float32
    )


def _classifier_body(vec_ref, wct_ref, bc_ref, wit_ref, bi_ref,
                     vect_ref, catt_ref, idt_ref):
    vec = vec_ref[:, 0:64]

    @pl.when(pl.program_id(0) == 0)
    def _():
        D = vec.shape[1]
        rows = lax.broadcasted_iota(jnp.int32, (D, D), 0)
        cols = lax.broadcasted_iota(jnp.int32, (D, D), 1)
        eye = (rows == cols).astype(jnp.float32)
        vect_ref[...] = _contract(eye, vec)
        catt_ref[...] = _contract(wct_ref[...], vec) + bc_ref[...]

    bi_col = jnp.swapaxes(bi_ref[...].reshape(1, -1), 0, 1)
    idt_ref[...] = _contract(wit_ref[...], vec) + bi_col


def _tc_classifiers(vec128, Wc, bc, Wi, bi):
    B = vec128.shape[0]
    C, D = Wc.shape
    V = Wi.shape[0]
    nblk = pl.cdiv(V, _VB)
    vect, catt, idt = pl.pallas_call(
        _classifier_body,
        grid=(nblk,),
        in_specs=[
            pl.BlockSpec((B, 128), lambda j: (0, 0)),
            pl.BlockSpec((D, C), lambda j: (0, 0)),
            pl.BlockSpec((C, 1), lambda j: (0, 0)),
            pl.BlockSpec((D, _VB), lambda j: (0, j)),
            pl.BlockSpec((_VB,), lambda j: (j,)),
        ],
        out_specs=[
            pl.BlockSpec((D, B), lambda j: (0, 0)),
            pl.BlockSpec((C, B), lambda j: (0, 0)),
            pl.BlockSpec((_VB, B), lambda j: (j, 0)),
        ],
        out_shape=[
            jax.ShapeDtypeStruct((D, B), jnp.float32),
            jax.ShapeDtypeStruct((C, B), jnp.float32),
            jax.ShapeDtypeStruct((V, B), jnp.float32),
        ],
    )(vec128, Wc.T, bc.reshape(C, 1), Wi.T, bi)
    return vect.T, catt.T, idt.T


def kernel(table, Wc, bc, Wi, bi, input_id):
    vec128 = _sc_gather(_tc_table_widen(table), input_id)
    vec_out, out_category, out_identity = _tc_classifiers(vec128, Wc, bc, Wi, bi)
    return (vec_out, out_category, out_identity)
